# baseline (speedup 1.0000x reference)
import functools

import jax
import jax.numpy as jnp
from jax import lax
from jax.experimental import pallas as pl
from jax.experimental.pallas import tpu as pltpu
from jax.experimental.pallas import tpu_sc as plsc

N = 100000
NPAD = 100352
C_IN = 16
C_OUT = 16
N_FORE = 50000
N_MID = N_FORE * 26
N_OUT = N_FORE * 28
EPS = 1e-3
SENT = 0x3FFFFFFF

TCBLK = 1024

NT = 16
CHUNK = NPAD // NT
VPC = CHUNK // 16
BINS = 1024
SCT = CHUNK // 128
GC = 3125
GPAD = 3200
GNG = GPAD // 128
ZROWS = N_MID // 32
ZCH = 325
NZDMA = ZROWS // ZCH


def _tc_body(f_ref, w_ref, scale_ref, beta_ref, sig_ref, r_ref, key_ref):
    f = f_ref[...]
    r = jnp.dot(f, w_ref[...], preferred_element_type=jnp.float32)
    r = r * scale_ref[...] + beta_ref[...]
    r_ref[...] = jnp.maximum(r, 0.0)
    sig = sig_ref[...]
    kb = lax.bitcast_convert_type(sig, jnp.int32)
    key = 0x3F800000 - kb
    i = pl.program_id(0)
    row = (i * TCBLK
           + lax.broadcasted_iota(jnp.int32, (8, 128), 0) * 128
           + lax.broadcasted_iota(jnp.int32, (8, 128), 1))
    key_ref[...] = jnp.where(row < N, key, SENT)


def _tc_transform(feats_pad, w_conv, scale, beta, sig_pad):
    grid = NPAD // TCBLK
    return pl.pallas_call(
        _tc_body,
        grid=(grid,),
        in_specs=[
            pl.BlockSpec((TCBLK, C_IN), lambda i: (i, 0)),
            pl.BlockSpec((C_IN, C_OUT), lambda i: (0, 0)),
            pl.BlockSpec((1, C_OUT), lambda i: (0, 0)),
            pl.BlockSpec((1, C_OUT), lambda i: (0, 0)),
            pl.BlockSpec((8, 128), lambda i: (i, 0)),
        ],
        out_specs=[
            pl.BlockSpec((TCBLK, C_OUT), lambda i: (i, 0)),
            pl.BlockSpec((8, 128), lambda i: (i, 0)),
        ],
        out_shape=[
            jax.ShapeDtypeStruct((NPAD, C_OUT), jnp.float32),
            jax.ShapeDtypeStruct((NPAD // 128, 128), jnp.int32),
        ],
    )(feats_pad, w_conv, scale, beta, sig_pad)


def _digit(k, shift):
    return lax.shift_right_logical(k, shift) & (BINS - 1)


def _sc_body(keys_hbm, r_hbm, out_hbm,
             key_a, idx_a, key_b, idx_b, ghist,
             zbuf, kbuf, vbuf, dbuf, hist, gh_all, gidx, gbuf,
             sem_zero, sem_g):
    core = lax.axis_index("c")
    si = lax.axis_index("s")
    wid = core * NT + si

    zv = jnp.zeros((16,), jnp.float32)

    @pl.loop(0, ZCH)
    def _(i):
        zbuf[i, :] = zv

    zbuf2 = zbuf
    zbase = N_FORE + wid * ZROWS

    @pl.loop(0, NZDMA)
    def _(i):
        pltpu.make_async_copy(
            zbuf2, out_hbm.at[pl.ds(zbase + i * ZCH, ZCH)], sem_zero).start()

    @pl.when(core == 0)
    def _():
        cnt0, _ = plsc.scan_count(jnp.zeros((16,), jnp.int32))
        adj = jnp.min(cnt0)
        base = pl.multiple_of(si * CHUNK, 8)

        for p in range(3):
            shift = 10 * p
            src_k, src_v, dst_k, dst_v = [
                (keys_hbm, None, key_a, idx_a),
                (key_a, idx_a, key_b, idx_b),
                (key_b, idx_b, key_a, idx_a),
            ][p]
            pltpu.sync_copy(src_k.at[pl.ds(base, CHUNK)], kbuf)
            if src_v is None:
                @pl.loop(0, VPC)
                def _(j):
                    vbuf[pl.ds(j * 16, 16)] = (
                        base + j * 16 + lax.iota(jnp.int32, 16))
            else:
                pltpu.sync_copy(src_v.at[pl.ds(base, CHUNK)], vbuf)

            @pl.loop(0, BINS // 16)
            def _(g):
                hist[pl.ds(g * 16, 16)] = jnp.zeros((16,), jnp.int32)

            @pl.loop(0, VPC)
            def _(j):
                d = _digit(kbuf[pl.ds(j * 16, 16)], shift)
                cnt, last = plsc.scan_count(d)
                plsc.addupdate_scatter(hist, [d], cnt - adj + 1, mask=last)

            pltpu.sync_copy(hist, ghist.at[si])
            plsc.subcore_barrier()
            pltpu.sync_copy(ghist, gh_all)

            def grp(g, s_carry):
                tot = jnp.zeros((16,), jnp.int32)
                part = jnp.zeros((16,), jnp.int32)
                for t in range(NT):
                    row = gh_all[t, pl.ds(g * 16, 16)]
                    part = part + jnp.where(t < si, row, 0)
                    tot = tot + row
                excl = plsc.cumsum(tot) - tot + s_carry
                hist[pl.ds(g * 16, 16)] = excl + part
                return s_carry + jnp.sum(tot)

            lax.fori_loop(0, BINS // 16, grp, jnp.int32(0))

            @pl.loop(0, VPC)
            def _(j):
                d = _digit(kbuf[pl.ds(j * 16, 16)], shift)
                cnt, last = plsc.scan_count(d)
                bofs = plsc.load_gather(hist, [d])
                dbuf[lax.div(j, 8), pl.ds(lax.rem(j, 8) * 16, 16)] = (
                    bofs + cnt - adj)
                plsc.addupdate_scatter(hist, [d], cnt - adj + 1, mask=last)

            for i in range(SCT):
                pltpu.make_async_copy(
                    kbuf.at[pl.ds(i * 128, 128)], dst_k.at[dbuf.at[i]],
                    sem_g).start()
                pltpu.make_async_copy(
                    vbuf.at[pl.ds(i * 128, 128)], dst_v.at[dbuf.at[i]],
                    sem_g).start()
            for i in range(SCT):
                pltpu.make_async_copy(
                    kbuf.at[pl.ds(i * 128, 128)], dst_k.at[dbuf.at[i]],
                    sem_g).wait()
                pltpu.make_async_copy(
                    vbuf.at[pl.ds(i * 128, 128)], dst_v.at[dbuf.at[i]],
                    sem_g).wait()
            plsc.subcore_barrier()

        for cc in range(2):
            ch = si * 2 + cc
            start = ch * GC
            astart = pl.multiple_of((start // 8) * 8, 8)
            shiftr = start - astart
            pltpu.sync_copy(idx_a.at[pl.ds(astart, GPAD)], gidx)
            for gj in range(GNG):
                pltpu.make_async_copy(
                    r_hbm.at[gidx.at[pl.ds(gj * 128, 128)]],
                    gbuf.at[pl.ds(gj * 128, 128)], sem_g).start()
            for gj in range(GNG):
                pltpu.make_async_copy(
                    r_hbm.at[gidx.at[pl.ds(gj * 128, 128)]],
                    gbuf.at[pl.ds(gj * 128, 128)], sem_g).wait()
            outbase = jnp.where(ch < 16, start, N_MID + start)
            pltpu.sync_copy(gbuf.at[pl.ds(shiftr, GC)],
                            out_hbm.at[pl.ds(outbase, GC)])

    @pl.loop(0, NZDMA)
    def _(i):
        pltpu.make_async_copy(
            zbuf2, out_hbm.at[pl.ds(zbase + i * ZCH, ZCH)], sem_zero).wait()


_sc_sort_gather = functools.partial(
    pl.kernel,
    out_type=jax.ShapeDtypeStruct((N_OUT, C_OUT), jnp.float32),
    mesh=plsc.VectorSubcoreMesh(core_axis_name="c", subcore_axis_name="s"),
    compiler_params=pltpu.CompilerParams(needs_layout_passes=False,
                                         use_tc_tiling_on_sc=False),
    scratch_types=[
        pltpu.VMEM_SHARED((NPAD,), jnp.int32),
        pltpu.VMEM_SHARED((NPAD,), jnp.int32),
        pltpu.VMEM_SHARED((NPAD,), jnp.int32),
        pltpu.VMEM_SHARED((NPAD,), jnp.int32),
        pltpu.VMEM_SHARED((NT, BINS), jnp.int32),
        pltpu.VMEM((ZCH, C_OUT), jnp.float32),
        pltpu.VMEM((CHUNK,), jnp.int32),
        pltpu.VMEM((CHUNK,), jnp.int32),
        pltpu.VMEM((SCT, 128), jnp.int32),
        pltpu.VMEM((BINS,), jnp.int32),
        pltpu.VMEM((NT, BINS), jnp.int32),
        pltpu.VMEM((GPAD,), jnp.int32),
        pltpu.VMEM((GPAD, C_OUT), jnp.float32),
        pltpu.SemaphoreType.DMA,
        pltpu.SemaphoreType.DMA,
    ])(_sc_body)


def kernel(features, indices, W_imp, W_conv, gamma, beta):
    del indices
    imp = features @ W_imp
    sig = jax.nn.sigmoid(imp[:, -1])

    sig_pad = jnp.pad(sig, (0, NPAD - N)).reshape(NPAD // 128, 128)
    feats_pad = jnp.pad(features, ((0, NPAD - N), (0, 0)))
    scale = (gamma / jnp.sqrt(1.0 + EPS)).reshape(1, C_OUT)
    r_rows, keys = _tc_transform(feats_pad, W_conv, scale,
                                 beta.reshape(1, C_OUT), sig_pad)
    return _sc_sort_gather(keys.reshape(NPAD), r_rows)
